# baseline (speedup 1.0000x reference)
import functools
import jax
import jax.numpy as jnp
from jax import lax
from jax.experimental import pallas as pl
from jax.experimental.pallas import tpu as pltpu, tpu_sc as plsc

VOXEL_SIZE = 0.015
MIN_BETA = VOXEL_SIZE
T_STEP = 0.01
N_TABLE = 2560000
P1, P2, P3 = 73856093, 19349663, 83492791

_info = plsc.get_sparse_core_info()
_NC, _NS = _info.num_cores, _info.num_subcores
_NW = _NC * _NS
_S = 64
_CH = 128
_BS = 1024


def _sc_gather_kernel(n_samples):
    blocks_per_worker = n_samples // (_NW * _S)
    chunks = (_S * 8) // _CH

    mesh = plsc.VectorSubcoreMesh(core_axis_name="c", subcore_axis_name="s")
    out_t = jax.ShapeDtypeStruct((n_samples * 8, 128), jnp.float32)

    @functools.partial(
        pl.kernel, mesh=mesh, out_type=out_t,
        scratch_types=[
            pltpu.VMEM((_S * 8,), jnp.int32),
            pltpu.VMEM((_S * 8, 128), jnp.float32),
            pltpu.SemaphoreType.DMA,
        ],
    )
    def k(table_hbm, idx_hbm, g_hbm, idx_v, rows_v, sem):
        wid = lax.axis_index("s") * _NC + lax.axis_index("c")

        def block_body(blk, carry):
            base = pl.multiple_of((wid * blocks_per_worker + blk) * _S, _S)
            base8 = pl.multiple_of(base * 8, _S * 8)
            pltpu.sync_copy(idx_hbm.at[pl.ds(base8, _S * 8)], idx_v)
            copies = [
                pltpu.async_copy(table_hbm.at[idx_v.at[pl.ds(c * _CH, _CH)]],
                                 rows_v.at[pl.ds(c * _CH, _CH)], sem)
                for c in range(chunks)
            ]
            for cp in copies:
                cp.wait()
            pltpu.sync_copy(rows_v, g_hbm.at[pl.ds(base8, _S * 8)])
            return carry

        lax.fori_loop(0, blocks_per_worker, block_body, 0)

    return k


def _tc_interp_kernel(g_ref, m_ref, fx_ref, fy_ref, fz_ref, out_ref):
    fx = fx_ref[...]
    fy = fy_ref[...]
    fz = fz_ref[...]
    wx = (1.0 - fx, fx)
    wy = (1.0 - fy, fy)
    wz = (1.0 - fz, fz)
    n = fx.shape[0]
    emb16 = jnp.zeros((n, 16), jnp.float32)
    gx = jnp.zeros_like(fx)
    gy = jnp.zeros_like(fx)
    gz = jnp.zeros_like(fx)
    for j in range(8):
        dx, dy, dz = (j >> 2) & 1, (j >> 1) & 1, j & 1
        gj = g_ref[:, j * 128:(j + 1) * 128]
        mj = m_ref[:, j:j + 1]
        row16 = jnp.zeros((n, 16), jnp.float32)
        for kk in range(8):
            sel = (mj == kk).astype(jnp.float32)
            row16 = row16 + sel * gj[:, kk * 16:(kk + 1) * 16]
        wj = wx[dx] * wy[dy] * wz[dz]
        emb16 = emb16 + wj * row16
        v0 = row16[:, 0:1]
        wyz = wy[dy] * wz[dz]
        wxz = wx[dx] * wz[dz]
        wxy = wx[dx] * wy[dy]
        gx = gx + wyz * v0 if dx else gx - wyz * v0
        gy = gy + wxz * v0 if dy else gy - wxz * v0
        gz = gz + wxy * v0 if dz else gz - wxy * v0
    inv_v = jnp.float32(1.0 / VOXEL_SIZE)
    out_ref[...] = jnp.concatenate(
        [emb16[:, 0:5], gx * inv_v, gy * inv_v, gz * inv_v, emb16[:, 8:16]],
        axis=1)


def _tc_scan_kernel(e0, e1, e2, e3, e4, g0, g1, g2, tmid, istart, bsc,
                    s0, s1, s2, s3, s4, s5, s6, s7):
    R = e0.shape[0]
    b = bsc[0, 0]
    alpha = 1.0 / b
    rix = lax.broadcasted_iota(jnp.int32, (R, R), 0)
    cix = lax.broadcasted_iota(jnp.int32, (R, R), 1)
    U = (rix <= cix).astype(jnp.float32)
    Lst = (cix < rix).astype(jnp.float32)

    sdf = e0[...]
    sig = alpha * (0.5 + 0.5 * jnp.sign(sdf)
                   * (jnp.exp(-jnp.abs(sdf) / b) - 1.0))
    m = (e4[...] > 0.0).astype(jnp.float32)
    sdt = sig * jnp.float32(T_STEP) * m

    def full_cumsum(x):
        incl = jnp.dot(x, U, preferred_element_type=jnp.float32,
                       precision=lax.Precision.HIGHEST)
        offs = jnp.dot(Lst, incl[:, R - 1:R],
                       preferred_element_type=jnp.float32,
                       precision=lax.Precision.HIGHEST)
        return incl + offs

    cs = full_cumsum(sdt)
    excl = cs - sdt

    neg = jnp.float32(-3.0e38)
    s = jnp.where(istart[...] > 0.0, excl, neg)
    mm = s
    k = 1
    while k < R:
        sh = jnp.concatenate(
            [jnp.full((R, k), neg, jnp.float32), mm[:, :R - k]], axis=1)
        mm = jnp.maximum(mm, sh)
        k *= 2
    rowmax = mm[:, R - 1:R]
    prev = jnp.concatenate(
        [jnp.full((1, 1), neg, jnp.float32), rowmax[:R - 1, :]], axis=0)
    k = 1
    while k < R:
        sh = jnp.concatenate(
            [jnp.full((k, 1), neg, jnp.float32), prev[:R - k, :]], axis=0)
        prev = jnp.maximum(prev, sh)
        k *= 2
    segoff = jnp.maximum(mm, prev)
    excl_in = excl - segoff
    T = jnp.exp(-excl_in)
    w = T * (1.0 - jnp.exp(-sdt))

    ga, gb, gc = g0[...], g1[...], g2[...]
    nrm = jnp.sqrt(ga * ga + gb * gb + gc * gc) + jnp.float32(1e-12)
    inv_n = w / nrm
    chans = (w, w * e1[...], w * e2[...], w * e3[...], w * tmid[...],
             inv_n * ga, inv_n * gb, inv_n * gc)
    souts = (s0, s1, s2, s3, s4, s5, s6, s7)
    for c in range(8):
        souts[c][...] = full_cumsum(chans[c])


def kernel(rays_o, rays_d, rays_d_norm, near, far, ray_indices, table, beta):
    n_rays = rays_o.shape[0]
    N = ray_indices.shape[0]
    R = 512

    ri = ray_indices.astype(jnp.int32)
    ar = jnp.arange(n_rays, dtype=jnp.int32)
    first_idx = jnp.searchsorted(ri, ar, side="left").astype(jnp.int32)
    end_idx = jnp.searchsorted(ri, ar, side="right").astype(jnp.int32)
    pos = jnp.arange(N, dtype=jnp.int32) - first_idx[ri]
    t_nears = near + pos.astype(jnp.float32) * T_STEP
    t_mid = 0.5 * (t_nears + (t_nears + T_STEP))
    x = rays_o[ri] + t_mid[:, None] * rays_d[ri]
    gq = x / VOXEL_SIZE
    g0f = jnp.floor(gq)
    f = gq - g0f
    gi = g0f.astype(jnp.int32)

    corner_idx = []
    for j in range(8):
        dx, dy, dz = (j >> 2) & 1, (j >> 1) & 1, j & 1
        h = (((gi[:, 0] + dx) * P1) ^ ((gi[:, 1] + dy) * P2)
             ^ ((gi[:, 2] + dz) * P3))
        corner_idx.append(jnp.mod(h, N_TABLE))
    idx8 = jnp.stack(corner_idx, axis=1)
    idx_fold = (idx8 // 8).reshape(-1)
    m8 = (idx8 % 8).astype(jnp.int32)
    table_fold = jnp.pad(table, ((0, 0), (0, 11))).reshape(N_TABLE // 8, 128)

    g_rows = _sc_gather_kernel(N)(table_fold, idx_fold)
    g2 = g_rows.reshape(N, 1024)

    nb = N // _BS
    O = pl.pallas_call(
        _tc_interp_kernel,
        grid=(nb,),
        in_specs=[
            pl.BlockSpec((_BS, 1024), lambda i: (i, 0)),
            pl.BlockSpec((_BS, 8), lambda i: (i, 0)),
            pl.BlockSpec((_BS, 1), lambda i: (i, 0)),
            pl.BlockSpec((_BS, 1), lambda i: (i, 0)),
            pl.BlockSpec((_BS, 1), lambda i: (i, 0)),
        ],
        out_specs=pl.BlockSpec((_BS, 16), lambda i: (i, 0)),
        out_shape=jax.ShapeDtypeStruct((N, 16), jnp.float32),
    )(g2, m8, f[:, 0:1], f[:, 1:2], f[:, 2:3])

    bsc = (MIN_BETA + jnp.abs(beta)).reshape(1, 1)
    istart = jnp.concatenate(
        [jnp.ones((1,), jnp.float32),
         (ri[1:] != ri[:-1]).astype(jnp.float32)])
    sq = lambda a: a.reshape(R, R)
    outs = pl.pallas_call(
        _tc_scan_kernel,
        out_shape=[jax.ShapeDtypeStruct((R, R), jnp.float32)
                   for _ in range(8)],
    )(sq(O[:, 0]), sq(O[:, 1]), sq(O[:, 2]), sq(O[:, 3]), sq(O[:, 4]),
      sq(O[:, 5]), sq(O[:, 6]), sq(O[:, 7]),
      sq(t_mid), sq(istart), bsc)

    zero = jnp.zeros((1,), jnp.float32)
    per_ray = []
    for c in range(8):
        Sp = jnp.concatenate([zero, outs[c].reshape(-1)])
        per_ray.append(Sp[end_idx] - Sp[first_idx])
    acc = per_ray[0][:, None]
    rgb = jnp.stack(per_ray[1:4], axis=1)
    depth = per_ray[4][:, None] / rays_d_norm
    nrm_o = jnp.stack(per_ray[5:8], axis=1)
    sdf_grads = O[:, 5:8]
    nears = jnp.full((n_rays, 1), near, dtype=jnp.float32) / rays_d_norm
    fars = jnp.full((n_rays, 1), far, dtype=jnp.float32) / rays_d_norm
    return rgb, depth, nrm_o, acc, sdf_grads, nears, fars


# speedup vs baseline: 1.4290x; 1.4290x over previous
import functools
import jax
import jax.numpy as jnp
from jax import lax
from jax.experimental import pallas as pl
from jax.experimental.pallas import tpu as pltpu, tpu_sc as plsc

VOXEL_SIZE = 0.015
MIN_BETA = VOXEL_SIZE
T_STEP = 0.01
N_TABLE = 2560000
P1, P2, P3 = 73856093, 19349663, 83492791

_info = plsc.get_sparse_core_info()
_NC, _NS = _info.num_cores, _info.num_subcores
_NW = _NC * _NS
_S = 64
_CH = 128
_BS = 1024


def _sc_gather_kernel(n_samples):
    blocks_per_worker = n_samples // (_NW * _S)
    chunks = (_S * 8) // _CH

    mesh = plsc.VectorSubcoreMesh(core_axis_name="c", subcore_axis_name="s")
    out_t = jax.ShapeDtypeStruct((n_samples * 8, 128), jnp.float32)

    @functools.partial(
        pl.kernel, mesh=mesh, out_type=out_t,
        scratch_types=[
            pltpu.VMEM((_S * 8,), jnp.int32),
            pltpu.VMEM((_S * 8, 128), jnp.float32),
            pltpu.SemaphoreType.DMA,
        ],
    )
    def k(table_hbm, idx_hbm, g_hbm, idx_v, rows_v, sem):
        wid = lax.axis_index("s") * _NC + lax.axis_index("c")

        def block_body(blk, carry):
            base = pl.multiple_of((wid * blocks_per_worker + blk) * _S, _S)
            base8 = pl.multiple_of(base * 8, _S * 8)
            pltpu.sync_copy(idx_hbm.at[pl.ds(base8, _S * 8)], idx_v)
            copies = [
                pltpu.async_copy(table_hbm.at[idx_v.at[pl.ds(c * _CH, _CH)]],
                                 rows_v.at[pl.ds(c * _CH, _CH)], sem)
                for c in range(chunks)
            ]
            for cp in copies:
                cp.wait()
            pltpu.sync_copy(rows_v, g_hbm.at[pl.ds(base8, _S * 8)])
            return carry

        lax.fori_loop(0, blocks_per_worker, block_body, 0)

    return k


def _tc_interp_kernel(g_ref, fx_ref, fy_ref, fz_ref, out_ref):
    fx = fx_ref[...]
    fy = fy_ref[...]
    fz = fz_ref[...]
    wx = (1.0 - fx, fx)
    wy = (1.0 - fy, fy)
    wz = (1.0 - fz, fz)
    n = fx.shape[0]
    emb16 = jnp.zeros((n, 16), jnp.float32)
    gx = jnp.zeros_like(fx)
    gy = jnp.zeros_like(fx)
    gz = jnp.zeros_like(fx)
    for j in range(8):
        dx, dy, dz = (j >> 2) & 1, (j >> 1) & 1, j & 1
        row16 = g_ref[:, j * 128:j * 128 + 16]
        wj = wx[dx] * wy[dy] * wz[dz]
        emb16 = emb16 + wj * row16
        v0 = row16[:, 0:1]
        wyz = wy[dy] * wz[dz]
        wxz = wx[dx] * wz[dz]
        wxy = wx[dx] * wy[dy]
        gx = gx + wyz * v0 if dx else gx - wyz * v0
        gy = gy + wxz * v0 if dy else gy - wxz * v0
        gz = gz + wxy * v0 if dz else gz - wxy * v0
    inv_v = jnp.float32(1.0 / VOXEL_SIZE)
    out_ref[...] = jnp.concatenate(
        [emb16[:, 0:5], gx * inv_v, gy * inv_v, gz * inv_v, emb16[:, 8:16]],
        axis=1)


def _tc_scan_kernel(e0, e1, e2, e3, e4, g0, g1, g2, tmid, istart, bsc,
                    s0, s1, s2, s3, s4, s5, s6, s7):
    R = e0.shape[0]
    b = bsc[0, 0]
    alpha = 1.0 / b
    rix = lax.broadcasted_iota(jnp.int32, (R, R), 0)
    cix = lax.broadcasted_iota(jnp.int32, (R, R), 1)
    U = (rix <= cix).astype(jnp.float32)
    Lst = (cix < rix).astype(jnp.float32)

    sdf = e0[...]
    sig = alpha * (0.5 + 0.5 * jnp.sign(sdf)
                   * (jnp.exp(-jnp.abs(sdf) / b) - 1.0))
    m = (e4[...] > 0.0).astype(jnp.float32)
    sdt = sig * jnp.float32(T_STEP) * m

    def full_cumsum(x):
        incl = jnp.dot(x, U, preferred_element_type=jnp.float32,
                       precision=lax.Precision.HIGHEST)
        offs = jnp.dot(Lst, incl[:, R - 1:R],
                       preferred_element_type=jnp.float32,
                       precision=lax.Precision.HIGHEST)
        return incl + offs

    cs = full_cumsum(sdt)
    excl = cs - sdt

    neg = jnp.float32(-3.0e38)
    s = jnp.where(istart[...] > 0.0, excl, neg)
    mm = s
    k = 1
    while k < R:
        sh = jnp.concatenate(
            [jnp.full((R, k), neg, jnp.float32), mm[:, :R - k]], axis=1)
        mm = jnp.maximum(mm, sh)
        k *= 2
    rowmax = mm[:, R - 1:R]
    prev = jnp.concatenate(
        [jnp.full((1, 1), neg, jnp.float32), rowmax[:R - 1, :]], axis=0)
    k = 1
    while k < R:
        sh = jnp.concatenate(
            [jnp.full((k, 1), neg, jnp.float32), prev[:R - k, :]], axis=0)
        prev = jnp.maximum(prev, sh)
        k *= 2
    segoff = jnp.maximum(mm, prev)
    excl_in = excl - segoff
    T = jnp.exp(-excl_in)
    w = T * (1.0 - jnp.exp(-sdt))

    ga, gb, gc = g0[...], g1[...], g2[...]
    nrm = jnp.sqrt(ga * ga + gb * gb + gc * gc) + jnp.float32(1e-12)
    inv_n = w / nrm
    chans = (w, w * e1[...], w * e2[...], w * e3[...], w * tmid[...],
             inv_n * ga, inv_n * gb, inv_n * gc)
    souts = (s0, s1, s2, s3, s4, s5, s6, s7)
    for c in range(8):
        souts[c][...] = full_cumsum(chans[c])


def kernel(rays_o, rays_d, rays_d_norm, near, far, ray_indices, table, beta):
    n_rays = rays_o.shape[0]
    N = ray_indices.shape[0]
    R = 512

    ri = ray_indices.astype(jnp.int32)
    ar = jnp.arange(n_rays, dtype=jnp.int32)
    first_idx = jnp.searchsorted(ri, ar, side="left").astype(jnp.int32)
    end_idx = jnp.searchsorted(ri, ar, side="right").astype(jnp.int32)
    pos = jnp.arange(N, dtype=jnp.int32) - first_idx[ri]
    t_nears = near + pos.astype(jnp.float32) * T_STEP
    t_mid = 0.5 * (t_nears + (t_nears + T_STEP))
    x = rays_o[ri] + t_mid[:, None] * rays_d[ri]
    gq = x / VOXEL_SIZE
    g0f = jnp.floor(gq)
    f = gq - g0f
    gi = g0f.astype(jnp.int32)

    corner_idx = []
    for j in range(8):
        dx, dy, dz = (j >> 2) & 1, (j >> 1) & 1, j & 1
        h = (((gi[:, 0] + dx) * P1) ^ ((gi[:, 1] + dy) * P2)
             ^ ((gi[:, 2] + dz) * P3))
        corner_idx.append(jnp.mod(h, N_TABLE))
    idx8 = jnp.stack(corner_idx, axis=1)
    idx_fold = idx8.reshape(-1)
    table_fold = jnp.pad(table, ((0, 0), (0, 123)))

    g_rows = _sc_gather_kernel(N)(table_fold, idx_fold)
    g2 = g_rows.reshape(N, 1024)

    nb = N // _BS
    O = pl.pallas_call(
        _tc_interp_kernel,
        grid=(nb,),
        in_specs=[
            pl.BlockSpec((_BS, 1024), lambda i: (i, 0)),
            pl.BlockSpec((_BS, 1), lambda i: (i, 0)),
            pl.BlockSpec((_BS, 1), lambda i: (i, 0)),
            pl.BlockSpec((_BS, 1), lambda i: (i, 0)),
        ],
        out_specs=pl.BlockSpec((_BS, 16), lambda i: (i, 0)),
        out_shape=jax.ShapeDtypeStruct((N, 16), jnp.float32),
    )(g2, f[:, 0:1], f[:, 1:2], f[:, 2:3])

    bsc = (MIN_BETA + jnp.abs(beta)).reshape(1, 1)
    istart = jnp.concatenate(
        [jnp.ones((1,), jnp.float32),
         (ri[1:] != ri[:-1]).astype(jnp.float32)])
    sq = lambda a: a.reshape(R, R)
    outs = pl.pallas_call(
        _tc_scan_kernel,
        out_shape=[jax.ShapeDtypeStruct((R, R), jnp.float32)
                   for _ in range(8)],
    )(sq(O[:, 0]), sq(O[:, 1]), sq(O[:, 2]), sq(O[:, 3]), sq(O[:, 4]),
      sq(O[:, 5]), sq(O[:, 6]), sq(O[:, 7]),
      sq(t_mid), sq(istart), bsc)

    zero = jnp.zeros((1,), jnp.float32)
    per_ray = []
    for c in range(8):
        Sp = jnp.concatenate([zero, outs[c].reshape(-1)])
        per_ray.append(Sp[end_idx] - Sp[first_idx])
    acc = per_ray[0][:, None]
    rgb = jnp.stack(per_ray[1:4], axis=1)
    depth = per_ray[4][:, None] / rays_d_norm
    nrm_o = jnp.stack(per_ray[5:8], axis=1)
    sdf_grads = O[:, 5:8]
    nears = jnp.full((n_rays, 1), near, dtype=jnp.float32) / rays_d_norm
    fars = jnp.full((n_rays, 1), far, dtype=jnp.float32) / rays_d_norm
    return rgb, depth, nrm_o, acc, sdf_grads, nears, fars
